# baseline (speedup 1.0000x reference)
import functools

import jax
import jax.numpy as jnp
from jax import lax
from jax.experimental import pallas as pl
from jax.experimental.pallas import tpu as pltpu

_VMEM_LIMIT_BYTES = 64 * 1024 * 1024


def _gn_nhwc_kernel(x_ref, g_ref, b_ref, o_ref, *, eps, m, cg):
    nb, _, _, c = x_ref.shape
    x = x_ref[...]
    xx = x * x
    s = jnp.sum(x, axis=1, keepdims=True)
    s = jnp.sum(s, axis=2, keepdims=True)
    ss = jnp.sum(xx, axis=1, keepdims=True)
    ss = jnp.sum(ss, axis=2, keepdims=True)

    v = jnp.concatenate([s.reshape(nb, c), ss.reshape(nb, c)], axis=0)
    ci = lax.broadcasted_iota(jnp.int32, (c, c), 0) // cg
    cj = lax.broadcasted_iota(jnp.int32, (c, c), 1) // cg
    mask = (ci == cj).astype(jnp.float32)
    gv = jnp.dot(v, mask, preferred_element_type=jnp.float32)

    gs = gv[0:nb, :]
    gss = gv[nb:2 * nb, :]
    mean = gs * (1.0 / m)
    var = jnp.maximum(gss - gs * mean, 0.0) * (1.0 / (m - 1))
    inv = pl.reciprocal(jnp.sqrt(var) + jnp.float32(eps), approx=False)
    scale = g_ref[...] * inv
    bias = b_ref[...] - mean * scale
    o_ref[...] = (x * scale.reshape(nb, 1, 1, c)
                  + bias.reshape(nb, 1, 1, c))


def _group_norm_2d(x, gamma, beta, *, group_num, eps):
    n, c, h, w = x.shape
    g = group_num
    cg = c // g
    m = cg * h * w

    x_t = jnp.transpose(x, (0, 2, 3, 1))
    gamma_r = gamma.reshape(1, c)
    beta_r = beta.reshape(1, c)

    nb = 2 if n % 2 == 0 else 1
    fused = functools.partial(_gn_nhwc_kernel, eps=float(eps), m=m, cg=cg)
    out_t = pl.pallas_call(
        fused,
        out_shape=jax.ShapeDtypeStruct((n, h, w, c), x.dtype),
        grid=(n // nb,),
        in_specs=[
            pl.BlockSpec((nb, h, w, c), lambda ni: (ni, 0, 0, 0)),
            pl.BlockSpec((1, c), lambda ni: (0, 0)),
            pl.BlockSpec((1, c), lambda ni: (0, 0)),
        ],
        out_specs=pl.BlockSpec((nb, h, w, c), lambda ni: (ni, 0, 0, 0)),
        compiler_params=pltpu.CompilerParams(
            dimension_semantics=("parallel",),
            vmem_limit_bytes=_VMEM_LIMIT_BYTES,
        ),
    )(x_t, gamma_r, beta_r)
    return jnp.transpose(out_t, (0, 3, 1, 2))


def kernel(x, gamma, beta):
    return _group_norm_2d(x, gamma, beta, group_num=32, eps=1e-10)
